# Initial kernel scaffold; baseline (speedup 1.0000x reference)
import jax
import jax.numpy as jnp
from jax.experimental import pallas as pl


def kernel(x, edge_index, edge_attr, batch, W_lin0, b_lin0, W_e1, b_e1, W_e2, b_e2, W_root, b_conv, W_ih, W_hh, b_ih, b_hh, W_li, W_lh, b_li, b_lh, W_lin1, b_lin1, W_lin2, b_lin2):
    raise NotImplementedError("write your pallas kernel here")


# speedup vs baseline: 3.0603x; 3.0603x over previous
import functools

import jax
import jax.numpy as jnp
from jax import lax
from jax.experimental import pallas as pl
from jax.experimental.pallas import tpu as pltpu
from jax.experimental.pallas import tpu_sc as plsc

N = 10000
E = 160000
F_IN = 128
F_E = 16
D = 32
B = 64
STEPS = 3
JUMPS = 3

CH = 128
NCH = E // CH


def _lin0_body(x_ref, w_ref, b_ref, o_ref):
    o_ref[...] = jax.nn.relu(
        jnp.dot(x_ref[...], w_ref[...], preferred_element_type=jnp.float32)
        + b_ref[...])


def _lin0(x, w, b):
    return pl.pallas_call(
        _lin0_body,
        out_shape=jax.ShapeDtypeStruct((N, D), jnp.float32),
    )(x, w, b.reshape(1, D))


TE = 1000


def _msg_body(ea_ref, xj_ref, w1_ref, b1_ref, w2_ref, b2_ref, k_ref, o_ref):
    h = jax.nn.relu(
        jnp.dot(ea_ref[...], w1_ref[...], preferred_element_type=jnp.float32)
        + b1_ref[...])
    wp = jnp.dot(h, w2_ref[...], preferred_element_type=jnp.float32) + b2_ref[...]
    xj = xj_ref[...]
    xt = jnp.concatenate([xj] * D, axis=1)
    o_ref[...] = jnp.dot(wp * xt, k_ref[...], preferred_element_type=jnp.float32)


def _msg(ea, xj, w1, b1, w2p, b2p, ksum):
    grid = E // TE
    return pl.pallas_call(
        _msg_body,
        grid=(grid,),
        in_specs=[
            pl.BlockSpec((TE, F_E), lambda i: (i, 0)),
            pl.BlockSpec((TE, D), lambda i: (i, 0)),
            pl.BlockSpec((F_E, F_IN), lambda i: (0, 0)),
            pl.BlockSpec((1, F_IN), lambda i: (0, 0)),
            pl.BlockSpec((F_IN, D * D), lambda i: (0, 0)),
            pl.BlockSpec((1, D * D), lambda i: (0, 0)),
            pl.BlockSpec((D * D, D), lambda i: (0, 0)),
        ],
        out_specs=pl.BlockSpec((TE, D), lambda i: (i, 0)),
        out_shape=jax.ShapeDtypeStruct((E, D), jnp.float32),
    )(ea, xj, w1, b1.reshape(1, F_IN), w2p, b2p, ksum)


def _update_body(a0_ref, a1_ref, d0_ref, d1_ref, out_ref, h_ref,
                 wr_ref, bc_ref, wih_ref, bih_ref, whh_ref, bhh_ref, o_ref):
    deg = jnp.maximum(d0_ref[...] + d1_ref[...], 1.0)
    agg = (a0_ref[...] + a1_ref[...]) / deg
    out = out_ref[...]
    h = h_ref[...]
    m = jax.nn.relu(
        agg + jnp.dot(out, wr_ref[...], preferred_element_type=jnp.float32)
        + bc_ref[...])
    gi = jnp.dot(m, wih_ref[...], preferred_element_type=jnp.float32) + bih_ref[...]
    gh = jnp.dot(h, whh_ref[...], preferred_element_type=jnp.float32) + bhh_ref[...]
    r = jax.nn.sigmoid(gi[:, :D] + gh[:, :D])
    z = jax.nn.sigmoid(gi[:, D:2 * D] + gh[:, D:2 * D])
    n = jnp.tanh(gi[:, 2 * D:] + r * gh[:, 2 * D:])
    o_ref[...] = (1.0 - z) * n + z * h


def _update(a0, a1, d0, d1, out, h, wr, bc, wihT, bih, whhT, bhh):
    return pl.pallas_call(
        _update_body,
        out_shape=jax.ShapeDtypeStruct((N, D), jnp.float32),
    )(a0, a1, d0, d1, out, h, wr, bc.reshape(1, D),
      wihT, bih.reshape(1, 3 * D), whhT, bhh.reshape(1, 3 * D))


def _set2set_body(out_ref, batch_ref, wli_ref, bli_ref, wlh_ref, blh_ref,
                  w1_ref, b1_ref, w2_ref, b2_ref, o_ref):
    out = out_ref[...]
    bvec = batch_ref[...]
    cols = lax.broadcasted_iota(jnp.int32, (N, B), 1)
    mask = (bvec == cols)
    mf = mask.astype(jnp.float32)

    q_star = jnp.zeros((B, 2 * D), jnp.float32)
    hl = jnp.zeros((B, D), jnp.float32)
    cl = jnp.zeros((B, D), jnp.float32)
    for _ in range(STEPS):
        g = (jnp.dot(q_star, wli_ref[...], preferred_element_type=jnp.float32)
             + bli_ref[...]
             + jnp.dot(hl, wlh_ref[...], preferred_element_type=jnp.float32)
             + blh_ref[...])
        ig = jax.nn.sigmoid(g[:, :D])
        fg = jax.nn.sigmoid(g[:, D:2 * D])
        cg = jnp.tanh(g[:, 2 * D:3 * D])
        og = jax.nn.sigmoid(g[:, 3 * D:])
        cl = fg * cl + ig * cg
        hl = og * jnp.tanh(cl)
        q = hl
        qb = jnp.dot(mf, q, preferred_element_type=jnp.float32)
        e = jnp.sum(out * qb, axis=1, keepdims=True)
        emax = jnp.max(jnp.where(mask, e, -1e30), axis=0, keepdims=True)
        emax_b = jnp.sum(mf * emax, axis=1, keepdims=True)
        ee = jnp.exp(e - emax_b)
        denom = jnp.sum(mf * ee, axis=0, keepdims=True)
        denom_b = jnp.sum(mf * denom, axis=1, keepdims=True)
        a = ee / (denom_b + 1e-16)
        rvec = lax.dot_general(mf * a, out, (((0,), (0,)), ((), ())),
                               preferred_element_type=jnp.float32)
        q_star = jnp.concatenate([q, rvec], axis=1)
    res = jax.nn.relu(
        jnp.dot(q_star, w1_ref[...], preferred_element_type=jnp.float32)
        + b1_ref[...])
    o_ref[...] = jnp.dot(res, w2_ref[...], preferred_element_type=jnp.float32) \
        + b2_ref[...]


def _set2set(out, batch, wliT, bli, wlhT, blh, w1, b1, w2, b2):
    return pl.pallas_call(
        _set2set_body,
        out_shape=jax.ShapeDtypeStruct((B, 1), jnp.float32),
    )(out, batch.reshape(N, 1), wliT, bli.reshape(1, 4 * D),
      wlhT, blh.reshape(1, 4 * D), w1, b1.reshape(1, D), w2, b2.reshape(1, 1))


_NC = 2
_NS = 16
_NW = _NC * _NS
_NPT = N // _NS


def _gather_kernel(idx_hbm, table_hbm, out_hbm, idx_v, rows_v, sem):
    w = lax.axis_index("s") * _NC + lax.axis_index("c")
    base = NCH // _NW
    extra = NCH - base * _NW
    nmine = base + jnp.where(w < extra, 1, 0)

    def body(r, carry):
        c = r * _NW + w
        pltpu.sync_copy(idx_hbm.at[c], idx_v.at[0])
        pltpu.async_copy(table_hbm.at[idx_v.at[0]], rows_v.at[0], sem).wait()
        pltpu.sync_copy(rows_v.at[0], out_hbm.at[c])
        return carry

    lax.fori_loop(0, nmine, body, 0)


def _sc_gather(table, idx2d):
    mesh = plsc.VectorSubcoreMesh(core_axis_name="c", subcore_axis_name="s")
    f = pl.kernel(
        _gather_kernel,
        out_type=jax.ShapeDtypeStruct((NCH, CH, D), jnp.float32),
        mesh=mesh,
        scratch_types=[
            pltpu.VMEM((1, CH), jnp.int32),
            pltpu.VMEM((1, CH, D), jnp.float32),
            pltpu.SemaphoreType.DMA,
        ],
        compiler_params=pltpu.CompilerParams(use_tc_tiling_on_sc=False),
    )
    return f(idx2d, table).reshape(E, D)


def _scatter_kernel(idx_hbm, msg_hbm, zero_hbm, out_hbm, idx_v, rows_v, acc):
    c = lax.axis_index("c")
    s = lax.axis_index("s")
    pltpu.sync_copy(zero_hbm.at[pl.ds(s * _NPT, _NPT)],
                    acc.at[pl.ds(s * _NPT, _NPT)])
    plsc.subcore_barrier()

    half = NCH // _NC
    base = half // _NS
    extra = half - base * _NS
    nmine = base + jnp.where(s < extra, 1, 0)

    def body(r, carry):
        ch = c * half + r * _NS + s
        pltpu.sync_copy(idx_hbm.at[ch], idx_v.at[0])
        pltpu.sync_copy(msg_hbm.at[ch], rows_v.at[0])
        pltpu.sync_copy(rows_v.at[0], acc.at[idx_v.at[0]], add=True)
        return carry

    lax.fori_loop(0, nmine, body, 0)
    plsc.subcore_barrier()
    pltpu.sync_copy(acc.at[pl.ds(s * _NPT, _NPT)],
                    out_hbm.at[c].at[pl.ds(s * _NPT, _NPT)])


def _sc_scatter(dst2d, msg3d, zeros):
    mesh = plsc.VectorSubcoreMesh(core_axis_name="c", subcore_axis_name="s")
    f = pl.kernel(
        _scatter_kernel,
        out_type=jax.ShapeDtypeStruct((2, N, D), jnp.float32),
        mesh=mesh,
        scratch_types=[
            pltpu.VMEM((1, CH), jnp.int32),
            pltpu.VMEM((1, CH, D), jnp.float32),
            pltpu.VMEM_SHARED((N, D), jnp.float32),
        ],
        compiler_params=pltpu.CompilerParams(use_tc_tiling_on_sc=False),
    )
    return f(dst2d, msg3d, zeros)


def kernel(x, edge_index, edge_attr, batch, W_lin0, b_lin0, W_e1, b_e1,
           W_e2, b_e2, W_root, b_conv, W_ih, W_hh, b_ih, b_hh,
           W_li, W_lh, b_li, b_lh, W_lin1, b_lin1, W_lin2, b_lin2):
    src2d = edge_index[0].reshape(NCH, CH)
    dst2d = edge_index[1].reshape(NCH, CH)
    zeros = jnp.zeros((N, D), jnp.float32)
    ones3d = jnp.ones((NCH, CH, D), jnp.float32)

    w2p = W_e2.reshape(F_IN, D, D).transpose(0, 2, 1).reshape(F_IN, D * D)
    b2p = b_e2.reshape(1, D, D).transpose(0, 2, 1).reshape(1, D * D)
    ksum = jnp.kron(jnp.eye(D, dtype=jnp.float32),
                    jnp.ones((D, 1), jnp.float32))

    wihT = W_ih.T
    whhT = W_hh.T
    wliT = W_li.T
    wlhT = W_lh.T

    out = _lin0(x, W_lin0, b_lin0)
    h = out

    degp = _sc_scatter(dst2d, ones3d, zeros)
    d0, d1 = degp[0], degp[1]

    for _ in range(JUMPS):
        xj = _sc_gather(out, src2d)
        msg = _msg(edge_attr, xj, W_e1, b_e1, w2p, b2p, ksum)
        aggp = _sc_scatter(dst2d, msg.reshape(NCH, CH, D), zeros)
        out = _update(aggp[0], aggp[1], d0, d1, out, h,
                      W_root, b_conv, wihT, b_ih, whhT, b_hh)
        h = out

    res = _set2set(out, batch, wliT, b_li, wlhT, b_lh,
                   W_lin1, b_lin1, W_lin2, b_lin2)
    return res.reshape(-1)


# speedup vs baseline: 3.4792x; 1.1369x over previous
import functools

import jax
import jax.numpy as jnp
from jax import lax
from jax.experimental import pallas as pl
from jax.experimental.pallas import tpu as pltpu
from jax.experimental.pallas import tpu_sc as plsc

N = 10000
E = 160000
F_IN = 128
F_E = 16
D = 32
B = 64
STEPS = 3
JUMPS = 3

CH = 128
NCH = E // CH


def _lin0_body(x_ref, w_ref, b_ref, o_ref):
    o_ref[...] = jax.nn.relu(
        jnp.dot(x_ref[...], w_ref[...], preferred_element_type=jnp.float32)
        + b_ref[...])


def _lin0(x, w, b):
    return pl.pallas_call(
        _lin0_body,
        out_shape=jax.ShapeDtypeStruct((N, D), jnp.float32),
    )(x, w, b.reshape(1, D))


TE = 1000


def _msg_body(ea_ref, xj_ref, w1_ref, b1_ref, w2_ref, b2_ref, k_ref, o_ref):
    h = jax.nn.relu(
        jnp.dot(ea_ref[...], w1_ref[...], preferred_element_type=jnp.float32)
        + b1_ref[...])
    wp = jnp.dot(h.astype(jnp.bfloat16), w2_ref[...],
                 preferred_element_type=jnp.float32) + b2_ref[...]
    xj = xj_ref[...]
    xt = jnp.concatenate([xj] * D, axis=1)
    prod = (wp * xt).astype(jnp.bfloat16)
    o_ref[...] = jnp.dot(prod, k_ref[...], preferred_element_type=jnp.float32)


def _msg(ea, xj, w1, b1, w2p, b2p, ksum):
    grid = E // TE
    return pl.pallas_call(
        _msg_body,
        grid=(grid,),
        in_specs=[
            pl.BlockSpec((TE, F_E), lambda i: (i, 0)),
            pl.BlockSpec((TE, D), lambda i: (i, 0)),
            pl.BlockSpec((F_E, F_IN), lambda i: (0, 0)),
            pl.BlockSpec((1, F_IN), lambda i: (0, 0)),
            pl.BlockSpec((F_IN, D * D), lambda i: (0, 0)),
            pl.BlockSpec((1, D * D), lambda i: (0, 0)),
            pl.BlockSpec((D * D, D), lambda i: (0, 0)),
        ],
        out_specs=pl.BlockSpec((TE, D), lambda i: (i, 0)),
        out_shape=jax.ShapeDtypeStruct((E, D), jnp.float32),
    )(ea, xj, w1, b1.reshape(1, F_IN), w2p, b2p, ksum)


def _update_body(a0_ref, a1_ref, d0_ref, d1_ref, out_ref, h_ref,
                 wr_ref, bc_ref, wih_ref, bih_ref, whh_ref, bhh_ref, o_ref):
    deg = jnp.maximum(d0_ref[...] + d1_ref[...], 1.0)
    agg = (a0_ref[...] + a1_ref[...]) / deg
    out = out_ref[...]
    h = h_ref[...]
    m = jax.nn.relu(
        agg + jnp.dot(out, wr_ref[...], preferred_element_type=jnp.float32)
        + bc_ref[...])
    gi = jnp.dot(m, wih_ref[...], preferred_element_type=jnp.float32) + bih_ref[...]
    gh = jnp.dot(h, whh_ref[...], preferred_element_type=jnp.float32) + bhh_ref[...]
    r = jax.nn.sigmoid(gi[:, :D] + gh[:, :D])
    z = jax.nn.sigmoid(gi[:, D:2 * D] + gh[:, D:2 * D])
    n = jnp.tanh(gi[:, 2 * D:] + r * gh[:, 2 * D:])
    o_ref[...] = (1.0 - z) * n + z * h


def _update(a0, a1, d0, d1, out, h, wr, bc, wihT, bih, whhT, bhh):
    return pl.pallas_call(
        _update_body,
        out_shape=jax.ShapeDtypeStruct((N, D), jnp.float32),
    )(a0, a1, d0, d1, out, h, wr, bc.reshape(1, D),
      wihT, bih.reshape(1, 3 * D), whhT, bhh.reshape(1, 3 * D))


def _set2set_body(out_ref, batch_ref, wli_ref, bli_ref, wlh_ref, blh_ref,
                  w1_ref, b1_ref, w2_ref, b2_ref, o_ref):
    out = out_ref[...]
    bvec = batch_ref[...]
    cols = lax.broadcasted_iota(jnp.int32, (N, B), 1)
    mask = (bvec == cols)
    mf = mask.astype(jnp.float32)

    q_star = jnp.zeros((B, 2 * D), jnp.float32)
    hl = jnp.zeros((B, D), jnp.float32)
    cl = jnp.zeros((B, D), jnp.float32)
    for _ in range(STEPS):
        g = (jnp.dot(q_star, wli_ref[...], preferred_element_type=jnp.float32)
             + bli_ref[...]
             + jnp.dot(hl, wlh_ref[...], preferred_element_type=jnp.float32)
             + blh_ref[...])
        ig = jax.nn.sigmoid(g[:, :D])
        fg = jax.nn.sigmoid(g[:, D:2 * D])
        cg = jnp.tanh(g[:, 2 * D:3 * D])
        og = jax.nn.sigmoid(g[:, 3 * D:])
        cl = fg * cl + ig * cg
        hl = og * jnp.tanh(cl)
        q = hl
        qb = jnp.dot(mf, q, preferred_element_type=jnp.float32)
        e = jnp.sum(out * qb, axis=1, keepdims=True)
        emax = jnp.max(jnp.where(mask, e, -1e30), axis=0, keepdims=True)
        emax_b = jnp.sum(mf * emax, axis=1, keepdims=True)
        ee = jnp.exp(e - emax_b)
        denom = jnp.sum(mf * ee, axis=0, keepdims=True)
        denom_b = jnp.sum(mf * denom, axis=1, keepdims=True)
        a = ee / (denom_b + 1e-16)
        rvec = lax.dot_general(mf * a, out, (((0,), (0,)), ((), ())),
                               preferred_element_type=jnp.float32)
        q_star = jnp.concatenate([q, rvec], axis=1)
    res = jax.nn.relu(
        jnp.dot(q_star, w1_ref[...], preferred_element_type=jnp.float32)
        + b1_ref[...])
    o_ref[...] = jnp.dot(res, w2_ref[...], preferred_element_type=jnp.float32) \
        + b2_ref[...]


def _set2set(out, batch, wliT, bli, wlhT, blh, w1, b1, w2, b2):
    return pl.pallas_call(
        _set2set_body,
        out_shape=jax.ShapeDtypeStruct((B, 1), jnp.float32),
    )(out, batch.reshape(N, 1), wliT, bli.reshape(1, 4 * D),
      wlhT, blh.reshape(1, 4 * D), w1, b1.reshape(1, D), w2, b2.reshape(1, 1))


_NC = 2
_NS = 16
_NW = _NC * _NS
_NPT = N // _NS


GK = 8
MAXR = 40
NGR = 5


def _gather_kernel(idx_hbm, table_hbm, out_hbm, idx_v, rows_v,
                   gs0, gs1, os0, os1):
    w = lax.axis_index("s") * _NC + lax.axis_index("c")
    base = NCH // _NW
    extra = NCH - base * _NW
    cnt = base + jnp.where(w < extra, 1, 0)
    start = w * base + jnp.minimum(w, extra)

    pltpu.sync_copy(idx_hbm.at[pl.ds(start, base)], idx_v.at[pl.ds(0, base)])

    @pl.when(w < extra)
    def _():
        pltpu.sync_copy(idx_hbm.at[pl.ds(start + base, 1)],
                        idx_v.at[pl.ds(base, 1)])

    gsem = (gs0, gs1)
    osem = (os0, os1)
    for g in range(NGR):
        hb = g % 2
        lo = hb * GK
        if g >= 2:
            for j in range(GK):
                r2 = (g - 2) * GK + j
                pltpu.make_async_copy(
                    rows_v.at[lo + j], out_hbm.at[start + r2],
                    osem[hb]).wait()
        for j in range(GK):
            r = g * GK + j

            @pl.when(r < cnt)
            def _(r=r, j=j):
                pltpu.async_copy(table_hbm.at[idx_v.at[r]],
                                 rows_v.at[lo + j], gsem[hb])
        for j in range(GK):
            r = g * GK + j

            @pl.when(r < cnt)
            def _(r=r, j=j):
                pltpu.make_async_copy(table_hbm.at[idx_v.at[r]],
                                      rows_v.at[lo + j], gsem[hb]).wait()
                pltpu.async_copy(rows_v.at[lo + j], out_hbm.at[start + r],
                                 osem[hb])
    for g in (NGR - 2, NGR - 1):
        hb = g % 2
        lo = hb * GK
        for j in range(GK):
            r = g * GK + j

            @pl.when(r < cnt)
            def _(r=r, j=j):
                pltpu.make_async_copy(
                    rows_v.at[lo + j], out_hbm.at[start + r],
                    osem[hb]).wait()


def _sc_gather(table, idx2d):
    mesh = plsc.VectorSubcoreMesh(core_axis_name="c", subcore_axis_name="s")
    f = pl.kernel(
        _gather_kernel,
        out_type=jax.ShapeDtypeStruct((NCH, CH, D), jnp.float32),
        mesh=mesh,
        scratch_types=[
            pltpu.VMEM((MAXR, CH), jnp.int32),
            pltpu.VMEM((2 * GK, CH, D), jnp.float32),
            pltpu.SemaphoreType.DMA,
            pltpu.SemaphoreType.DMA,
            pltpu.SemaphoreType.DMA,
            pltpu.SemaphoreType.DMA,
        ],
        compiler_params=pltpu.CompilerParams(use_tc_tiling_on_sc=False),
    )
    return f(idx2d, table).reshape(E, D)


def _scatter_kernel(idx_hbm, msg_hbm, zero_hbm, out_hbm, idx_v, rows_v, acc,
                    ls0, ls1, ss0, ss1):
    c = lax.axis_index("c")
    s = lax.axis_index("s")
    pltpu.sync_copy(zero_hbm.at[pl.ds(s * _NPT, _NPT)],
                    acc.at[pl.ds(s * _NPT, _NPT)])
    plsc.subcore_barrier()

    half = NCH // _NC
    base = half // _NS
    extra = half - base * _NS
    cnt = base + jnp.where(s < extra, 1, 0)
    start = c * half + s * base + jnp.minimum(s, extra)

    pltpu.sync_copy(idx_hbm.at[pl.ds(start, base)], idx_v.at[pl.ds(0, base)])

    @pl.when(s < extra)
    def _():
        pltpu.sync_copy(idx_hbm.at[pl.ds(start + base, 1)],
                        idx_v.at[pl.ds(base, 1)])

    lsem = (ls0, ls1)
    ssem = (ss0, ss1)
    for g in range(NGR):
        hb = g % 2
        lo = hb * GK
        if g >= 2:
            for j in range(GK):
                r2 = (g - 2) * GK + j
                pltpu.make_async_copy(
                    rows_v.at[lo + j], acc.at[idx_v.at[r2]],
                    ssem[hb]).wait()
        for j in range(GK):
            r = g * GK + j

            @pl.when(r < cnt)
            def _(r=r, j=j):
                pltpu.async_copy(msg_hbm.at[start + r], rows_v.at[lo + j],
                                 lsem[hb])
        for j in range(GK):
            r = g * GK + j

            @pl.when(r < cnt)
            def _(r=r, j=j):
                pltpu.make_async_copy(msg_hbm.at[start + r],
                                      rows_v.at[lo + j], lsem[hb]).wait()
                pltpu.async_copy(rows_v.at[lo + j], acc.at[idx_v.at[r]],
                                 ssem[hb], add=True)
    for g in (NGR - 2, NGR - 1):
        hb = g % 2
        lo = hb * GK
        for j in range(GK):
            r = g * GK + j

            @pl.when(r < cnt)
            def _(r=r, j=j):
                pltpu.make_async_copy(
                    rows_v.at[lo + j], acc.at[idx_v.at[r]],
                    ssem[hb]).wait()

    plsc.subcore_barrier()
    pltpu.sync_copy(acc.at[pl.ds(s * _NPT, _NPT)],
                    out_hbm.at[c].at[pl.ds(s * _NPT, _NPT)])


def _sc_scatter(dst2d, msg3d, zeros):
    mesh = plsc.VectorSubcoreMesh(core_axis_name="c", subcore_axis_name="s")
    f = pl.kernel(
        _scatter_kernel,
        out_type=jax.ShapeDtypeStruct((2, N, D), jnp.float32),
        mesh=mesh,
        scratch_types=[
            pltpu.VMEM((MAXR, CH), jnp.int32),
            pltpu.VMEM((2 * GK, CH, D), jnp.float32),
            pltpu.VMEM_SHARED((N, D), jnp.float32),
            pltpu.SemaphoreType.DMA,
            pltpu.SemaphoreType.DMA,
            pltpu.SemaphoreType.DMA,
            pltpu.SemaphoreType.DMA,
        ],
        compiler_params=pltpu.CompilerParams(use_tc_tiling_on_sc=False),
    )
    return f(dst2d, msg3d, zeros)


def kernel(x, edge_index, edge_attr, batch, W_lin0, b_lin0, W_e1, b_e1,
           W_e2, b_e2, W_root, b_conv, W_ih, W_hh, b_ih, b_hh,
           W_li, W_lh, b_li, b_lh, W_lin1, b_lin1, W_lin2, b_lin2):
    src2d = edge_index[0].reshape(NCH, CH)
    dst2d = edge_index[1].reshape(NCH, CH)
    zeros = jnp.zeros((N, D), jnp.float32)
    ones3d = jnp.ones((NCH, CH, D), jnp.float32)

    w2p = W_e2.reshape(F_IN, D, D).transpose(0, 2, 1).reshape(F_IN, D * D)
    b2p = b_e2.reshape(1, D, D).transpose(0, 2, 1).reshape(1, D * D)
    ksum = jnp.kron(jnp.eye(D, dtype=jnp.float32),
                    jnp.ones((D, 1), jnp.float32)).astype(jnp.bfloat16)
    w2p = w2p.astype(jnp.bfloat16)

    wihT = W_ih.T
    whhT = W_hh.T
    wliT = W_li.T
    wlhT = W_lh.T

    out = _lin0(x, W_lin0, b_lin0)
    h = out

    degp = _sc_scatter(dst2d, ones3d, zeros)
    d0, d1 = degp[0], degp[1]

    for _ in range(JUMPS):
        xj = _sc_gather(out, src2d)
        msg = _msg(edge_attr, xj, W_e1, b_e1, w2p, b2p, ksum)
        aggp = _sc_scatter(dst2d, msg.reshape(NCH, CH, D), zeros)
        out = _update(aggp[0], aggp[1], d0, d1, out, h,
                      W_root, b_conv, wihT, b_ih, whhT, b_hh)
        h = out

    res = _set2set(out, batch, wliT, b_li, wlhT, b_lh,
                   W_lin1, b_lin1, W_lin2, b_lin2)
    return res.reshape(-1)
